# Initial kernel scaffold; baseline (speedup 1.0000x reference)
import jax
import jax.numpy as jnp
from jax.experimental import pallas as pl


def kernel(x, embed_table, W, b):
    raise NotImplementedError("write your pallas kernel here")


# speedup vs baseline: 3.3888x; 3.3888x over previous
import functools

import jax
import jax.numpy as jnp
from jax import lax
from jax.experimental import pallas as pl
from jax.experimental.pallas import tpu as pltpu
from jax.experimental.pallas import tpu_sc as plsc

VOCAB = 100000
HID = 128
NCLS = 128
B = 4096
L = 50

NC = 2
NS = 16
NW = NC * NS
BPW = B // NW
NCH = NCLS // 16

ROWS_BLK = 2000


def _transform_body(t_ref, w_ref, b_ref, z_ref):
    z = jnp.dot(t_ref[...], w_ref[...], preferred_element_type=jnp.float32)
    z_ref[...] = jnp.maximum(z + b_ref[...], 0.0)


def _transform_table(table, W, b):
    return pl.pallas_call(
        _transform_body,
        grid=(VOCAB // ROWS_BLK,),
        in_specs=[
            pl.BlockSpec((ROWS_BLK, HID), lambda i: (i, 0)),
            pl.BlockSpec((HID, NCLS), lambda i: (0, 0)),
            pl.BlockSpec((1, NCLS), lambda i: (0, 0)),
        ],
        out_specs=pl.BlockSpec((ROWS_BLK, NCLS), lambda i: (i, 0)),
        out_shape=jax.ShapeDtypeStruct((VOCAB, NCLS), jnp.float32),
    )(table, W, b.reshape(1, NCLS))


def _softmax_inplace(buf):
    def sum_body(l, accs):
        out = []
        for c in range(NCH):
            e = jnp.exp(buf[l, pl.ds(c * 16, 16)])
            buf[l, pl.ds(c * 16, 16)] = e
            out.append(accs[c] + e)
        return tuple(out)

    zeros = tuple(jnp.zeros((16,), jnp.float32) for _ in range(NCH))
    accs = lax.fori_loop(0, L, sum_body, zeros)
    invs = tuple(1.0 / a for a in accs)

    def norm_body(l, carry):
        for c in range(NCH):
            buf[l, pl.ds(c * 16, 16)] = buf[l, pl.ds(c * 16, 16)] * invs[c]
        return carry

    lax.fori_loop(0, L, norm_body, 0)


def _gather_softmax_kernel(x_hbm, z_hbm, out_hbm, idx_v, buf0, buf1,
                           sem0, sem1):
    wid = lax.axis_index("s") * NC + lax.axis_index("c")
    base = wid * BPW
    pltpu.sync_copy(x_hbm.at[pl.ds(base, BPW), :], idx_v)

    def gather(i, buf, sem):
        return pltpu.make_async_copy(z_hbm.at[idx_v.at[i]], buf, sem)

    def process(i, buf, sem):
        gather(i, buf, sem).wait()
        _softmax_inplace(buf)
        pltpu.sync_copy(buf, out_hbm.at[base + i])

    gather(0, buf0, sem0).start()

    def body(i2, carry):
        i = i2 * 2
        gather(i + 1, buf1, sem1).start()
        process(i, buf0, sem0)

        @pl.when(i2 + 1 < BPW // 2)
        def _():
            gather(i + 2, buf0, sem0).start()

        process(i + 1, buf1, sem1)
        return carry

    lax.fori_loop(0, BPW // 2, body, 0)


@functools.cache
def _gather_softmax():
    return pl.kernel(
        _gather_softmax_kernel,
        out_type=jax.ShapeDtypeStruct((B, L, NCLS), jnp.float32),
        mesh=plsc.VectorSubcoreMesh(core_axis_name="c", subcore_axis_name="s"),
        scratch_types=[
            pltpu.VMEM((BPW, L), jnp.int32),
            pltpu.VMEM((L, NCLS), jnp.float32),
            pltpu.VMEM((L, NCLS), jnp.float32),
            pltpu.SemaphoreType.DMA,
            pltpu.SemaphoreType.DMA,
        ],
    )


def kernel(x, embed_table, W, b):
    z = _transform_table(embed_table, W, b)
    return _gather_softmax()(x, z)


# speedup vs baseline: 3.6002x; 1.0624x over previous
import functools

import jax
import jax.numpy as jnp
from jax import lax
from jax.experimental import pallas as pl
from jax.experimental.pallas import tpu as pltpu
from jax.experimental.pallas import tpu_sc as plsc

VOCAB = 100000
HID = 128
NCLS = 128
B = 4096
L = 50

NC = 2
NS = 16
NW = NC * NS
BPW = B // NW
NCH = NCLS // 16

ROWS_BLK = 2000


def _transform_body(t_ref, w_ref, b_ref, z_ref):
    z = jnp.dot(t_ref[...], w_ref[...], preferred_element_type=jnp.float32)
    z_ref[...] = jnp.exp(jnp.maximum(z + b_ref[...], 0.0))


def _transform_table(table, W, b):
    return pl.pallas_call(
        _transform_body,
        grid=(VOCAB // ROWS_BLK,),
        in_specs=[
            pl.BlockSpec((ROWS_BLK, HID), lambda i: (i, 0)),
            pl.BlockSpec((HID, NCLS), lambda i: (0, 0)),
            pl.BlockSpec((1, NCLS), lambda i: (0, 0)),
        ],
        out_specs=pl.BlockSpec((ROWS_BLK, NCLS), lambda i: (i, 0)),
        out_shape=jax.ShapeDtypeStruct((VOCAB, NCLS), jnp.float32),
    )(table, W, b.reshape(1, NCLS))


def _softmax_inplace(buf):
    def sum_body(l, accs):
        return tuple(accs[c] + buf[l, pl.ds(c * 16, 16)]
                     for c in range(NCH))

    zeros = tuple(jnp.zeros((16,), jnp.float32) for _ in range(NCH))
    accs = lax.fori_loop(0, L, sum_body, zeros)
    invs = tuple(1.0 / a for a in accs)

    def norm_body(l, carry):
        for c in range(NCH):
            buf[l, pl.ds(c * 16, 16)] = buf[l, pl.ds(c * 16, 16)] * invs[c]
        return carry

    lax.fori_loop(0, L, norm_body, 0)


def _gather_softmax_kernel(x_hbm, z_hbm, out_hbm, idx_v, buf0, buf1,
                           sem0, sem1):
    wid = lax.axis_index("s") * NC + lax.axis_index("c")
    base = wid * BPW
    pltpu.sync_copy(x_hbm.at[pl.ds(base, BPW), :], idx_v)

    def gather(i, buf, sem):
        return pltpu.make_async_copy(z_hbm.at[idx_v.at[i]], buf, sem)

    def process(i, buf, sem):
        gather(i, buf, sem).wait()
        _softmax_inplace(buf)
        pltpu.sync_copy(buf, out_hbm.at[base + i])

    gather(0, buf0, sem0).start()

    def body(i2, carry):
        i = i2 * 2
        gather(i + 1, buf1, sem1).start()
        process(i, buf0, sem0)

        @pl.when(i2 + 1 < BPW // 2)
        def _():
            gather(i + 2, buf0, sem0).start()

        process(i + 1, buf1, sem1)
        return carry

    lax.fori_loop(0, BPW // 2, body, 0)


@functools.cache
def _gather_softmax():
    return pl.kernel(
        _gather_softmax_kernel,
        out_type=jax.ShapeDtypeStruct((B, L, NCLS), jnp.float32),
        mesh=plsc.VectorSubcoreMesh(core_axis_name="c", subcore_axis_name="s"),
        scratch_types=[
            pltpu.VMEM((BPW, L), jnp.int32),
            pltpu.VMEM((L, NCLS), jnp.float32),
            pltpu.VMEM((L, NCLS), jnp.float32),
            pltpu.SemaphoreType.DMA,
            pltpu.SemaphoreType.DMA,
        ],
    )


def kernel(x, embed_table, W, b):
    z = _transform_table(embed_table, W, b)
    return _gather_softmax()(x, z)
